# Initial kernel scaffold; baseline (speedup 1.0000x reference)
import jax
import jax.numpy as jnp
from jax.experimental import pallas as pl


def kernel(x, edge_index, W1, b1, W2, b2):
    raise NotImplementedError("write your pallas kernel here")


# speedup vs baseline: 15.5505x; 15.5505x over previous
import functools

import jax
import jax.numpy as jnp
from jax import lax
from jax.experimental import pallas as pl
from jax.experimental.pallas import tpu as pltpu
from jax.experimental.pallas import tpu_sc as plsc

NC = 2
NS = 16
NW = NC * NS
LANES = 16
CHUNK = 128
BLK = 512


def _mesh():
    return plsc.VectorSubcoreMesh(
        core_axis_name="c", subcore_axis_name="s",
        num_cores=NC, num_subcores=NS)


def _fill_1d(ref, n, value):
    def body(i, _):
        ref[pl.ds(i * LANES, LANES)] = jnp.full((LANES,), value, jnp.float32)
        return 0
    lax.fori_loop(0, n // LANES, body, 0)


def _fill_2d(ref, rows, cols, value):
    per_row = cols // LANES
    def body(i, _):
        r = i // per_row
        c = i % per_row
        ref[r, pl.ds(c * LANES, LANES)] = jnp.full((LANES,), value, jnp.float32)
        return 0
    lax.fori_loop(0, rows * per_row, body, 0)


def _make_deg_kernel(EP, NP):
    EPW = EP // NW
    n_chunks = EPW // CHUNK
    RPT = NP // NS

    @functools.partial(
        pl.kernel,
        out_type=jax.ShapeDtypeStruct((NC, NP), jnp.float32),
        mesh=_mesh(),
        scratch_types=[
            pltpu.VMEM((CHUNK,), jnp.int32),
            pltpu.VMEM((CHUNK,), jnp.float32),
            pltpu.VMEM((RPT,), jnp.float32),
            pltpu.VMEM_SHARED((NP,), jnp.float32),
        ],
    )
    def deg_kernel(dst_hbm, out_hbm, idx_v, ones_v, zbuf, acc_sh):
        cid = lax.axis_index("c")
        sid = lax.axis_index("s")
        wid = sid * NC + cid
        _fill_1d(ones_v, CHUNK, 1.0)
        _fill_1d(zbuf, RPT, 0.0)
        pltpu.sync_copy(zbuf, acc_sh.at[pl.ds(sid * RPT, RPT)])
        plsc.subcore_barrier()
        base = wid * EPW

        def chunk(j, _):
            pltpu.sync_copy(dst_hbm.at[pl.ds(base + j * CHUNK, CHUNK)], idx_v)
            pltpu.sync_copy(ones_v, acc_sh.at[idx_v], add=True)
            return 0
        lax.fori_loop(0, n_chunks, chunk, 0)
        plsc.subcore_barrier()
        pltpu.sync_copy(acc_sh.at[pl.ds(sid * RPT, RPT)],
                        out_hbm.at[cid, pl.ds(sid * RPT, RPT)])

    return deg_kernel


def _make_scatter_kernel(EP, NP, D):
    EPW = EP // NW
    n_chunks = EPW // CHUNK
    RPT = NP // NS
    ZR = 64

    @functools.partial(
        pl.kernel,
        out_type=jax.ShapeDtypeStruct((NC, NP, D), jnp.float32),
        mesh=_mesh(),
        scratch_types=[
            pltpu.VMEM((CHUNK,), jnp.int32),
            pltpu.VMEM((CHUNK,), jnp.int32),
            pltpu.VMEM((CHUNK, D), jnp.float32),
            pltpu.VMEM((ZR, D), jnp.float32),
            pltpu.VMEM_SHARED((NP, D), jnp.float32),
            pltpu.SemaphoreType.DMA,
        ],
    )
    def scat_kernel(hs_hbm, src_hbm, dst_hbm, out_hbm,
                    sidx, didx, rows, zbuf, acc_sh, sem):
        cid = lax.axis_index("c")
        sid = lax.axis_index("s")
        wid = sid * NC + cid
        _fill_2d(zbuf, ZR, D, 0.0)

        def zero_acc(k, _):
            pltpu.sync_copy(zbuf, acc_sh.at[pl.ds(sid * RPT + k * ZR, ZR)])
            return 0
        lax.fori_loop(0, RPT // ZR, zero_acc, 0)
        plsc.subcore_barrier()
        base = wid * EPW

        def chunk(j, _):
            pltpu.sync_copy(src_hbm.at[pl.ds(base + j * CHUNK, CHUNK)], sidx)
            pltpu.async_copy(hs_hbm.at[sidx], rows, sem).wait()
            pltpu.sync_copy(dst_hbm.at[pl.ds(base + j * CHUNK, CHUNK)], didx)
            pltpu.sync_copy(rows, acc_sh.at[didx], add=True)
            return 0
        lax.fori_loop(0, n_chunks, chunk, 0)
        plsc.subcore_barrier()
        pltpu.sync_copy(acc_sh.at[pl.ds(sid * RPT, RPT)],
                        out_hbm.at[cid, pl.ds(sid * RPT, RPT)])

    return scat_kernel


def _tc1_body(x_ref, w1_ref, degp_ref, hs1_ref, disb_ref):
    deg = degp_ref[0, :] + degp_ref[1, :] + 1.0
    dis = lax.rsqrt(deg)
    h = jnp.dot(x_ref[...], w1_ref[...], preferred_element_type=jnp.float32)
    hs1_ref[...] = h * dis[:, None]
    disb_ref[...] = jnp.broadcast_to(dis[:, None], hs1_ref.shape)


def _make_tc1(NP, Df, H):
    return pl.pallas_call(
        _tc1_body,
        grid=(NP // BLK,),
        in_specs=[
            pl.BlockSpec((BLK, Df), lambda i: (i, 0)),
            pl.BlockSpec((Df, H), lambda i: (0, 0)),
            pl.BlockSpec((NC, BLK), lambda i: (0, i)),
        ],
        out_specs=[
            pl.BlockSpec((BLK, H), lambda i: (i, 0)),
            pl.BlockSpec((BLK, H), lambda i: (i, 0)),
        ],
        out_shape=[
            jax.ShapeDtypeStruct((NP, H), jnp.float32),
            jax.ShapeDtypeStruct((NP, H), jnp.float32),
        ],
    )


def _tc2_body(p0_ref, p1_ref, hs1_ref, disb_ref, b1_ref, w2_ref, hs2_ref):
    t = (p0_ref[...] + p1_ref[...] + hs1_ref[...]) * disb_ref[...] + b1_ref[...]
    t = jnp.maximum(t, 0.0)
    h2 = jnp.dot(t, w2_ref[...], preferred_element_type=jnp.float32)
    hs2_ref[...] = h2 * disb_ref[:, : h2.shape[1]]


def _make_tc2(NP, H, C):
    return pl.pallas_call(
        _tc2_body,
        grid=(NP // BLK,),
        in_specs=[
            pl.BlockSpec((BLK, H), lambda i: (i, 0)),
            pl.BlockSpec((BLK, H), lambda i: (i, 0)),
            pl.BlockSpec((BLK, H), lambda i: (i, 0)),
            pl.BlockSpec((BLK, H), lambda i: (i, 0)),
            pl.BlockSpec((1, H), lambda i: (0, 0)),
            pl.BlockSpec((H, C), lambda i: (0, 0)),
        ],
        out_specs=pl.BlockSpec((BLK, C), lambda i: (i, 0)),
        out_shape=jax.ShapeDtypeStruct((NP, C), jnp.float32),
    )


def _tc3_body(q0_ref, q1_ref, hs2_ref, disb_ref, b2_ref, out_ref):
    agg = q0_ref[...] + q1_ref[...] + hs2_ref[...]
    out_ref[...] = agg * disb_ref[:, : agg.shape[1]] + b2_ref[...]


def _make_tc3(NP, H, C):
    return pl.pallas_call(
        _tc3_body,
        grid=(NP // BLK,),
        in_specs=[
            pl.BlockSpec((BLK, C), lambda i: (i, 0)),
            pl.BlockSpec((BLK, C), lambda i: (i, 0)),
            pl.BlockSpec((BLK, C), lambda i: (i, 0)),
            pl.BlockSpec((BLK, H), lambda i: (i, 0)),
            pl.BlockSpec((1, C), lambda i: (0, 0)),
        ],
        out_specs=pl.BlockSpec((BLK, C), lambda i: (i, 0)),
        out_shape=jax.ShapeDtypeStruct((NP, C), jnp.float32),
    )


def kernel(x, edge_index, W1, b1, W2, b2):
    N, Df = x.shape
    H = W1.shape[1]
    C = W2.shape[1]
    E = edge_index.shape[1]

    NP = -(-(N + LANES) // BLK) * BLK
    EP = -(-E // (NW * CHUNK)) * (NW * CHUNK)

    src = edge_index[0].astype(jnp.int32)
    dst = edge_index[1].astype(jnp.int32)
    n_pad = EP - E
    if n_pad:
        pad_idx = (N + (jnp.arange(n_pad, dtype=jnp.int32) % (NP - N)))
        src = jnp.concatenate([src, pad_idx])
        dst = jnp.concatenate([dst, pad_idx])
    x_p = jnp.concatenate(
        [x.astype(jnp.float32), jnp.zeros((NP - N, Df), jnp.float32)])

    CP = max(C, 128)
    W2p = jnp.concatenate(
        [W2, jnp.zeros((H, CP - C), jnp.float32)], axis=1) if CP != C else W2
    b2p = jnp.concatenate(
        [b2, jnp.zeros((CP - C,), jnp.float32)]) if CP != C else b2

    degp = _make_deg_kernel(EP, NP)(dst)
    hs1, disb = _make_tc1(NP, Df, H)(x_p, W1, degp)
    P = _make_scatter_kernel(EP, NP, H)(hs1, src, dst)
    hs2 = _make_tc2(NP, H, CP)(P[0], P[1], hs1, disb,
                               b1.reshape(1, H), W2p)
    Q = _make_scatter_kernel(EP, NP, CP)(hs2, src, dst)
    out_p = _make_tc3(NP, H, CP)(Q[0], Q[1], hs2, disb, b2p.reshape(1, CP))
    return out_p[:N, :C]


# speedup vs baseline: 27.1548x; 1.7462x over previous
import functools

import jax
import jax.numpy as jnp
from jax import lax
from jax.experimental import pallas as pl
from jax.experimental.pallas import tpu as pltpu
from jax.experimental.pallas import tpu_sc as plsc

NC = 2
NS = 16
NW = NC * NS
LANES = 16
CHUNK = 128
NBUF = 2
BLK = 512


def _mesh():
    return plsc.VectorSubcoreMesh(
        core_axis_name="c", subcore_axis_name="s",
        num_cores=NC, num_subcores=NS)


def _fill_1d(ref, n, value):
    def body(i, _):
        ref[pl.ds(i * LANES, LANES)] = jnp.full((LANES,), value, jnp.float32)
        return 0
    lax.fori_loop(0, n // LANES, body, 0)


def _fill_2d(ref, rows, cols, value):
    per_row = cols // LANES
    def body(i, _):
        r = i // per_row
        c = i % per_row
        ref[r, pl.ds(c * LANES, LANES)] = jnp.full((LANES,), value, jnp.float32)
        return 0
    lax.fori_loop(0, rows * per_row, body, 0)


def _make_deg_kernel(EP, NP):
    n_chunks = EP // (NW * CHUNK)
    RPT = NP // NS

    @functools.partial(
        pl.kernel,
        out_type=jax.ShapeDtypeStruct((NC, NP), jnp.float32),
        mesh=_mesh(),
        scratch_types=[
            pltpu.VMEM((n_chunks, CHUNK), jnp.int32),
            pltpu.VMEM((CHUNK,), jnp.float32),
            pltpu.VMEM((RPT,), jnp.float32),
            pltpu.VMEM_SHARED((NP,), jnp.float32),
            pltpu.SemaphoreType.DMA,
            pltpu.SemaphoreType.DMA,
        ],
    )
    def deg_kernel(dstm_hbm, out_hbm, idx_v, ones_v, zbuf, acc_sh,
                   sem_i, sem_s):
        cid = lax.axis_index("c")
        sid = lax.axis_index("s")
        wid = sid * NC + cid
        gbase = wid * n_chunks
        pltpu.async_copy(dstm_hbm.at[pl.ds(gbase, n_chunks)], idx_v, sem_i)
        _fill_1d(ones_v, CHUNK, 1.0)
        _fill_1d(zbuf, RPT, 0.0)
        pltpu.sync_copy(zbuf, acc_sh.at[pl.ds(sid * RPT, RPT)])
        pltpu.make_async_copy(
            dstm_hbm.at[pl.ds(0, n_chunks)], idx_v, sem_i).wait()
        plsc.subcore_barrier()

        def chunk(j, _):
            pltpu.async_copy(ones_v, acc_sh.at[idx_v.at[j]], sem_s, add=True)

            @pl.when(j >= 8)
            def _():
                pltpu.make_async_copy(
                    ones_v, acc_sh.at[pl.ds(0, CHUNK)], sem_s).wait()
            return 0
        lax.fori_loop(0, n_chunks, chunk, 0)

        def drain(j, _):
            pltpu.make_async_copy(
                ones_v, acc_sh.at[pl.ds(0, CHUNK)], sem_s).wait()
            return 0
        lax.fori_loop(0, min(8, n_chunks), drain, 0)
        plsc.subcore_barrier()
        pltpu.sync_copy(acc_sh.at[pl.ds(sid * RPT, RPT)],
                        out_hbm.at[cid, pl.ds(sid * RPT, RPT)])

    return deg_kernel


def _make_scatter_kernel(EP, NP, D):
    n_chunks = EP // (NW * CHUNK)
    n_outer = n_chunks // NBUF
    RPT = NP // NS
    ZR = 16

    @functools.partial(
        pl.kernel,
        out_type=jax.ShapeDtypeStruct((NC, NP, D), jnp.float32),
        mesh=_mesh(),
        scratch_types=[
            pltpu.VMEM((2, NBUF, 2, CHUNK), jnp.int32),
            pltpu.VMEM((NBUF, CHUNK, D), jnp.float32),
            pltpu.VMEM((ZR, D), jnp.float32),
            pltpu.VMEM_SHARED((NP, D), jnp.float32),
            pltpu.SemaphoreType.DMA((NBUF,)),
            pltpu.SemaphoreType.DMA((NBUF,)),
            pltpu.SemaphoreType.DMA((NBUF,)),
        ],
    )
    def scat_kernel(hs_hbm, srcm_hbm, dstm_hbm, out_hbm,
                    idxb, rows, zbuf, acc_sh, sem_i, sem_g, sem_s):
        cid = lax.axis_index("c")
        sid = lax.axis_index("s")
        wid = sid * NC + cid
        gbase = wid * n_chunks
        for b in range(NBUF):
            pltpu.async_copy(srcm_hbm.at[gbase + b], idxb.at[0, b, 0],
                             sem_i.at[b])
            pltpu.async_copy(dstm_hbm.at[gbase + b], idxb.at[1, b, 0],
                             sem_i.at[b])
        _fill_2d(zbuf, ZR, D, 0.0)

        def zero_acc(k, _):
            pltpu.sync_copy(zbuf, acc_sh.at[pl.ds(sid * RPT + k * ZR, ZR)])
            return 0
        lax.fori_loop(0, RPT // ZR, zero_acc, 0)
        plsc.subcore_barrier()

        def outer(g, _):
            p = g % 2
            q = 1 - p
            for b in range(NBUF):
                j = g * NBUF + b
                pltpu.make_async_copy(
                    srcm_hbm.at[gbase], idxb.at[0, b, p], sem_i.at[b]).wait()
                pltpu.make_async_copy(
                    srcm_hbm.at[gbase], idxb.at[1, b, p], sem_i.at[b]).wait()

                @pl.when(g > 0)
                def _wait_prev_scatter():
                    pltpu.make_async_copy(
                        rows.at[b], acc_sh.at[pl.ds(0, CHUNK)],
                        sem_s.at[b]).wait()

                @pl.when(g < n_outer - 1)
                def _prefetch_idx():
                    pltpu.async_copy(srcm_hbm.at[gbase + j + NBUF],
                                     idxb.at[0, b, q], sem_i.at[b])
                    pltpu.async_copy(dstm_hbm.at[gbase + j + NBUF],
                                     idxb.at[1, b, q], sem_i.at[b])

                pltpu.async_copy(
                    hs_hbm.at[idxb.at[0, b, p]], rows.at[b],
                    sem_g.at[b]).wait()
                pltpu.async_copy(
                    rows.at[b], acc_sh.at[idxb.at[1, b, p]], sem_s.at[b],
                    add=True)
            return 0
        lax.fori_loop(0, n_outer, outer, 0)

        for b in range(NBUF):
            pltpu.make_async_copy(
                rows.at[b], acc_sh.at[pl.ds(0, CHUNK)], sem_s.at[b]).wait()
        plsc.subcore_barrier()
        pltpu.sync_copy(acc_sh.at[pl.ds(sid * RPT, RPT)],
                        out_hbm.at[cid, pl.ds(sid * RPT, RPT)])

    return scat_kernel


def _tc1_body(x_ref, w1_ref, degp_ref, hs1_ref, disb_ref):
    deg = degp_ref[0, :] + degp_ref[1, :] + 1.0
    dis = lax.rsqrt(deg)
    h = jnp.dot(x_ref[...], w1_ref[...], preferred_element_type=jnp.float32)
    hs1_ref[...] = h * dis[:, None]
    disb_ref[...] = jnp.broadcast_to(dis[:, None], hs1_ref.shape)


def _make_tc1(NP, Df, H):
    return pl.pallas_call(
        _tc1_body,
        grid=(NP // BLK,),
        in_specs=[
            pl.BlockSpec((BLK, Df), lambda i: (i, 0)),
            pl.BlockSpec((Df, H), lambda i: (0, 0)),
            pl.BlockSpec((NC, BLK), lambda i: (0, i)),
        ],
        out_specs=[
            pl.BlockSpec((BLK, H), lambda i: (i, 0)),
            pl.BlockSpec((BLK, H), lambda i: (i, 0)),
        ],
        out_shape=[
            jax.ShapeDtypeStruct((NP, H), jnp.float32),
            jax.ShapeDtypeStruct((NP, H), jnp.float32),
        ],
    )


def _tc2_body(p0_ref, p1_ref, hs1_ref, disb_ref, b1_ref, w2_ref, hs2_ref):
    t = (p0_ref[...] + p1_ref[...] + hs1_ref[...]) * disb_ref[...] + b1_ref[...]
    t = jnp.maximum(t, 0.0)
    h2 = jnp.dot(t, w2_ref[...], preferred_element_type=jnp.float32)
    hs2_ref[...] = h2 * disb_ref[:, : h2.shape[1]]


def _make_tc2(NP, H, C):
    return pl.pallas_call(
        _tc2_body,
        grid=(NP // BLK,),
        in_specs=[
            pl.BlockSpec((BLK, H), lambda i: (i, 0)),
            pl.BlockSpec((BLK, H), lambda i: (i, 0)),
            pl.BlockSpec((BLK, H), lambda i: (i, 0)),
            pl.BlockSpec((BLK, H), lambda i: (i, 0)),
            pl.BlockSpec((1, H), lambda i: (0, 0)),
            pl.BlockSpec((H, C), lambda i: (0, 0)),
        ],
        out_specs=pl.BlockSpec((BLK, C), lambda i: (i, 0)),
        out_shape=jax.ShapeDtypeStruct((NP, C), jnp.float32),
    )


def _tc3_body(q0_ref, q1_ref, hs2_ref, disb_ref, b2_ref, out_ref):
    agg = q0_ref[...] + q1_ref[...] + hs2_ref[...]
    out_ref[...] = agg * disb_ref[:, : agg.shape[1]] + b2_ref[...]


def _make_tc3(NP, H, C):
    return pl.pallas_call(
        _tc3_body,
        grid=(NP // BLK,),
        in_specs=[
            pl.BlockSpec((BLK, C), lambda i: (i, 0)),
            pl.BlockSpec((BLK, C), lambda i: (i, 0)),
            pl.BlockSpec((BLK, C), lambda i: (i, 0)),
            pl.BlockSpec((BLK, H), lambda i: (i, 0)),
            pl.BlockSpec((1, C), lambda i: (0, 0)),
        ],
        out_specs=pl.BlockSpec((BLK, C), lambda i: (i, 0)),
        out_shape=jax.ShapeDtypeStruct((NP, C), jnp.float32),
    )


def kernel(x, edge_index, W1, b1, W2, b2):
    N, Df = x.shape
    H = W1.shape[1]
    C = W2.shape[1]
    E = edge_index.shape[1]

    NP = -(-(N + LANES) // BLK) * BLK
    EPQ = NW * CHUNK * NBUF
    EP = -(-E // EPQ) * EPQ

    src = edge_index[0].astype(jnp.int32)
    dst = edge_index[1].astype(jnp.int32)
    n_pad = EP - E
    if n_pad:
        pad_idx = (N + (jnp.arange(n_pad, dtype=jnp.int32) % (NP - N)))
        src = jnp.concatenate([src, pad_idx])
        dst = jnp.concatenate([dst, pad_idx])
    srcm = src.reshape(-1, CHUNK)
    dstm = dst.reshape(-1, CHUNK)
    x_p = jnp.concatenate(
        [x.astype(jnp.float32), jnp.zeros((NP - N, Df), jnp.float32)])

    CP = max(C, 128)
    W2p = jnp.concatenate(
        [W2, jnp.zeros((H, CP - C), jnp.float32)], axis=1) if CP != C else W2
    b2p = jnp.concatenate(
        [b2, jnp.zeros((CP - C,), jnp.float32)]) if CP != C else b2

    degp = _make_deg_kernel(EP, NP)(dstm)
    hs1, disb = _make_tc1(NP, Df, H)(x_p, W1, degp)
    P = _make_scatter_kernel(EP, NP, H)(hs1, srcm, dstm)
    hs2 = _make_tc2(NP, H, CP)(P[0], P[1], hs1, disb,
                               b1.reshape(1, H), W2p)
    Q = _make_scatter_kernel(EP, NP, CP)(hs2, srcm, dstm)
    out_p = _make_tc3(NP, H, CP)(Q[0], Q[1], hs2, disb, b2p.reshape(1, CP))
    return out_p[:N, :C]
